# baseline (speedup 1.0000x reference)
import functools
import math

import jax
import jax.numpy as jnp
from jax import lax
from jax.experimental import pallas as pl
from jax.experimental.pallas import tpu as pltpu
from jax.experimental.pallas import tpu_sc as plsc

N = 10000
E = 320000
F_IN = 128
HID = 50
HP = 64
WH = 16
NCLS = 4
K1 = int(math.ceil(0.5 * N))
K2 = int(math.ceil(0.5 * K1))
NPAD = 10240
NROW2D = NPAD // 128

NC = 2
NS = 16
NW = NC * NS
CH = 128
NCHUNK = E // CH
RPT = 632
ACCN = RPT * NS
KSS = 8
NCHUNKP = 2560
EP = NCHUNKP * CH
SSW = NCHUNKP // NW
NSSW = SSW // KSS

GB = 1000
NB = N // GB

_SC_MESH = plsc.VectorSubcoreMesh(core_axis_name="c", subcore_axis_name="s")


def _sc_gather_scatter_add(width):
    @functools.partial(
        pl.kernel,
        out_type=jax.ShapeDtypeStruct((NC, ACCN, width), jnp.float32),
        mesh=_SC_MESH,
        compiler_params=pltpu.CompilerParams(use_tc_tiling_on_sc=False),
        scratch_types=(
            [pltpu.VMEM((CH,), jnp.int32) for _ in range(2 * KSS)]
            + [
                pltpu.VMEM((KSS, CH, width), jnp.float32),
                pltpu.VMEM_SHARED((ACCN, width), jnp.float32),
                pltpu.SemaphoreType.DMA,
                pltpu.SemaphoreType.DMA,
            ]
        ),
    )
    def k(table, srcv, dstv, zrows, out, *scr):
        sidx = scr[0:KSS]
        didx = scr[KSS:2 * KSS]
        rows, acc, semg, sems = scr[2 * KSS:]
        c = lax.axis_index("c")
        s = lax.axis_index("s")
        wid = s * NC + c
        pltpu.sync_copy(zrows, acc.at[pl.ds(s * RPT, RPT)])
        plsc.subcore_barrier()
        base = wid * SSW * CH

        def step(i, carry):
            e0 = pl.multiple_of(base + i * (KSS * CH), CH)
            ld = [pltpu.async_copy(srcv.at[pl.ds(e0 + j * CH, CH)], sidx[j],
                                   semg) for j in range(KSS)]
            ld += [pltpu.async_copy(dstv.at[pl.ds(e0 + j * CH, CH)], didx[j],
                                    semg) for j in range(KSS)]
            for d in ld:
                d.wait()
            gd = [pltpu.async_copy(table.at[sidx[j]], rows.at[j], semg)
                  for j in range(KSS)]
            for d in gd:
                d.wait()
            for j in range(KSS):
                pltpu.sync_copy(rows.at[j], acc.at[didx[j]], add=True)
            return carry

        lax.fori_loop(0, NSSW, step, 0)
        plsc.subcore_barrier()
        pltpu.sync_copy(acc.at[pl.ds(s * RPT, RPT)],
                        out.at[c, pl.ds(s * RPT, RPT)])

    return k


def _sc_scatter_ones():
    @functools.partial(
        pl.kernel,
        out_type=jax.ShapeDtypeStruct((NC, ACCN, WH), jnp.float32),
        mesh=_SC_MESH,
        compiler_params=pltpu.CompilerParams(use_tc_tiling_on_sc=False),
        scratch_types=(
            [pltpu.VMEM((CH,), jnp.int32) for _ in range(KSS)]
            + [
                pltpu.VMEM((CH, WH), jnp.float32),
                pltpu.VMEM_SHARED((ACCN, WH), jnp.float32),
                pltpu.SemaphoreType.DMA,
                pltpu.SemaphoreType.DMA,
            ]
        ),
    )
    def k(dstv, onesr, zrows, out, *scr):
        didx = scr[0:KSS]
        rows, acc, semg, sems = scr[KSS:]
        c = lax.axis_index("c")
        s = lax.axis_index("s")
        wid = s * NC + c
        pltpu.sync_copy(zrows, acc.at[pl.ds(s * RPT, RPT)])
        pltpu.sync_copy(onesr, rows)
        plsc.subcore_barrier()
        base = wid * SSW * CH

        def step(i, carry):
            e0 = pl.multiple_of(base + i * (KSS * CH), CH)
            ld = [pltpu.async_copy(dstv.at[pl.ds(e0 + j * CH, CH)], didx[j],
                                   semg) for j in range(KSS)]
            for d in ld:
                d.wait()
            for j in range(KSS):
                pltpu.sync_copy(rows, acc.at[didx[j]], add=True)
            return carry

        lax.fori_loop(0, NSSW, step, 0)
        plsc.subcore_barrier()
        pltpu.sync_copy(acc.at[pl.ds(s * RPT, RPT)],
                        out.at[c, pl.ds(s * RPT, RPT)])

    return k


def _k12_body(x_ref, w_ref, hist_ref, xw_ref, xs_ref):
    xw = jnp.dot(x_ref[...], w_ref[...], preferred_element_type=jnp.float32)
    d = hist_ref[0] + hist_ref[1]
    dinv = lax.rsqrt(1.0 + d[:, 0:1])
    xw_ref[...] = xw
    xs_ref[...] = xw * dinv


def _k3a_body(a_ref, xw_ref, hist_ref, b_ref, p_ref, x1_ref, sc_ref):
    d = hist_ref[0] + hist_ref[1]
    dinv = lax.rsqrt(1.0 + d[:, 0:1])
    agg = a_ref[0] + a_ref[1]
    xw = xw_ref[...]
    x1 = dinv * agg + (dinv * dinv) * xw + b_ref[...]
    x1_ref[...] = x1
    pn = p_ref[...]
    pnorm2 = jnp.sum(pn[:, 0:1] * pn[:, 0:1])
    sc = jnp.dot(x1, pn, preferred_element_type=jnp.float32) * lax.rsqrt(pnorm2)
    sc_ref[...] = jnp.tanh(sc)


def _select_body(k_const, sc_ref, msk_ref, m_ref, v_ref):
    sc = sc_ref[...]
    ridx = lax.broadcasted_iota(jnp.int32, (NROW2D, 128), 0)
    cidx = lax.broadcasted_iota(jnp.int32, (NROW2D, 128), 1)
    idx = ridx * 128 + cidx
    valid = (idx < N) & (msk_ref[...] > 0)
    bits = lax.bitcast_convert_type(sc, jnp.uint32)
    key = jnp.where((bits >> 31) == 0, bits | jnp.uint32(0x80000000), ~bits)
    key = jnp.where(valid, key, jnp.uint32(0))

    def tstep(b, t):
        sh = lax.shift_left(jnp.uint32(1), jnp.uint32(31) - b.astype(jnp.uint32))
        t_try = t | sh
        c = jnp.sum((key >= t_try).astype(jnp.int32))
        return jnp.where(c >= k_const, t_try, t)

    tthr = lax.fori_loop(0, 32, tstep, jnp.uint32(0))
    gt = key > tthr
    c_gt = jnp.sum(gt.astype(jnp.int32))
    r = k_const - c_gt
    eq = key == tthr

    def jstep(b, j):
        j_try = j | lax.shift_left(jnp.int32(1), jnp.int32(14) - b)
        c = jnp.sum((eq & (idx < j_try)).astype(jnp.int32))
        return jnp.where(c <= r, j_try, j)

    jcut = lax.fori_loop(0, 15, jstep, jnp.int32(0))
    sel = gt | (eq & (idx < jcut))
    m_ref[...] = sel.astype(jnp.float32)
    v_ref[...] = jnp.where(sel, sc, 0.0)


def _k3c_body(x1_ref, sel_ref, m_ref, w2_ref, xw2_ref, g_ref):
    i = pl.program_id(0)
    y = x1_ref[...] * sel_ref[...][:, 0:1]
    xw2_ref[...] = jnp.dot(y, w2_ref[...], preferred_element_type=jnp.float32)
    mcol = m_ref[...][:, 0:1] > 0
    bmax = jnp.max(jnp.where(mcol, y, -jnp.inf), axis=0, keepdims=True)
    bsum = jnp.sum(jnp.where(mcol, y, 0.0), axis=0, keepdims=True)

    @pl.when(i == 0)
    def _():
        g_ref[...] = jnp.concatenate([bmax, bsum], axis=0)

    @pl.when(i > 0)
    def _():
        prev = g_ref[...]
        g_ref[...] = jnp.concatenate(
            [jnp.maximum(prev[0:1], bmax), prev[1:2] + bsum], axis=0)

    @pl.when(i == NB - 1)
    def _():
        g = g_ref[...]
        g_ref[...] = jnp.concatenate([g[0:1], g[1:2] * (1.0 / K1)], axis=0)


def _k4_body(xw2_ref, hist_ref, m_ref, xs2_ref, dinv_ref):
    d = hist_ref[0] + hist_ref[1]
    mcol = m_ref[...][:, 0:1] > 0
    dinv = jnp.where(mcol, lax.rsqrt(1.0 + d[:, 0:1]), 0.0)
    xs2_ref[...] = xw2_ref[...] * dinv
    dinv_ref[...] = jnp.broadcast_to(dinv, (GB, 8))


def _k5a_body(a_ref, xw2_ref, dinv_ref, b_ref, p_ref, x2_ref, sc_ref):
    dinv = dinv_ref[...][:, 0:1]
    agg = a_ref[0] + a_ref[1]
    xw2 = xw2_ref[...]
    x2 = dinv * agg + (dinv * dinv) * xw2 + b_ref[...]
    x2_ref[...] = x2
    pn = p_ref[...]
    pnorm2 = jnp.sum(pn[:, 0:1] * pn[:, 0:1])
    sc = jnp.dot(x2, pn, preferred_element_type=jnp.float32) * lax.rsqrt(pnorm2)
    sc_ref[...] = jnp.tanh(sc)


def _k5c_body(x2_ref, sel_ref, m_ref, g_ref):
    i = pl.program_id(0)
    y = x2_ref[...] * sel_ref[...][:, 0:1]
    mcol = m_ref[...][:, 0:1] > 0
    bmax = jnp.max(jnp.where(mcol, y, -jnp.inf), axis=0, keepdims=True)
    bsum = jnp.sum(jnp.where(mcol, y, 0.0), axis=0, keepdims=True)

    @pl.when(i == 0)
    def _():
        g_ref[...] = jnp.concatenate([bmax, bsum], axis=0)

    @pl.when(i > 0)
    def _():
        prev = g_ref[...]
        g_ref[...] = jnp.concatenate(
            [jnp.maximum(prev[0:1], bmax), prev[1:2] + bsum], axis=0)

    @pl.when(i == NB - 1)
    def _():
        g = g_ref[...]
        g_ref[...] = jnp.concatenate([g[0:1], g[1:2] * (1.0 / K2)], axis=0)


def _k6_body(g1_ref, g2_ref, wfc_ref, bfc_ref, out_ref):
    dot = functools.partial(jnp.dot, preferred_element_type=jnp.float32)
    logits = (dot(g1_ref[0:1], wfc_ref[0]) + dot(g1_ref[1:2], wfc_ref[1])
              + dot(g2_ref[0:1], wfc_ref[2]) + dot(g2_ref[1:2], wfc_ref[3]))
    logits = logits + bfc_ref[...]
    col = lax.broadcasted_iota(jnp.int32, (1, 128), 1)
    neg = jnp.where(col < NCLS, logits, -jnp.inf)
    m = jnp.max(neg)
    e = jnp.where(col < NCLS, jnp.exp(logits - m), 0.0)
    lse = jnp.log(jnp.sum(e)) + m
    out_ref[...] = jnp.broadcast_to(logits - lse, (8, 128))


def _rb(width):
    return pl.BlockSpec((GB, width), lambda i: (i, 0))


def _pb(shape):
    return pl.BlockSpec(shape, lambda i: tuple(0 for _ in shape))


def _hb(width):
    return pl.BlockSpec((NC, GB, width), lambda i: (0, i, 0))


def _f32(*shape):
    return jax.ShapeDtypeStruct(shape, jnp.float32)


def _pad2d(flat8):
    return jnp.pad(flat8[:, 0], (0, NPAD - N)).reshape(NROW2D, 128)


def _torep(arr2d):
    flat = arr2d.reshape(NPAD)[:N]
    return jnp.broadcast_to(flat[:, None], (N, 8))


def kernel(x, edge_index, batch, W1, b1, W2, b2, p1, p2, Wfc, bfc):
    f32 = jnp.float32
    src = edge_index[0].astype(jnp.int32)
    dst = edge_index[1].astype(jnp.int32)
    sink = 10016 + (jnp.arange(EP - E, dtype=jnp.int32) % 64)
    srcp = jnp.concatenate([src, jnp.zeros((EP - E,), jnp.int32)])
    dstp = jnp.concatenate([dst, sink])

    W1p = jnp.zeros((F_IN, HP), f32).at[:, :HID].set(W1)
    W2p = jnp.zeros((HP, HP), f32).at[:HID, :HID].set(W2)
    b1p = jnp.zeros((1, HP), f32).at[0, :HID].set(b1)
    b2p = jnp.zeros((1, HP), f32).at[0, :HID].set(b2)
    p1rep = jnp.broadcast_to(
        jnp.zeros((HP,), f32).at[:HID].set(p1)[:, None], (HP, 8))
    p2rep = jnp.broadcast_to(
        jnp.zeros((HP,), f32).at[:HID].set(p2)[:, None], (HP, 8))
    wfc_pad = jnp.zeros((4, HP, 128), f32)
    for blk in range(4):
        wfc_pad = wfc_pad.at[blk, :HID, :NCLS].set(Wfc[blk * HID:(blk + 1) * HID])
    bfc_pad = jnp.zeros((1, 128), f32).at[0, :NCLS].set(bfc)
    zrows_h = jnp.zeros((RPT, WH), f32)
    zrows_f = jnp.zeros((RPT, HP), f32)
    ones_r = jnp.ones((CH, WH), f32)
    ones2d = jnp.ones((NROW2D, 128), f32)

    hist1 = _sc_scatter_ones()(dstp, ones_r, zrows_h)

    xw1, xs1 = pl.pallas_call(
        _k12_body,
        grid=(NB,),
        in_specs=[_rb(F_IN), _pb((F_IN, HP)), _hb(WH)],
        out_specs=[_rb(HP), _rb(HP)],
        out_shape=[_f32(N, HP), _f32(N, HP)],
    )(x, W1p, hist1)

    A1 = _sc_gather_scatter_add(HP)(xs1, srcp, dstp, zrows_f)

    x1, sc1 = pl.pallas_call(
        _k3a_body,
        grid=(NB,),
        in_specs=[_hb(HP), _rb(HP), _hb(WH), _pb((1, HP)), _pb((HP, 8))],
        out_specs=[_rb(HP), _rb(8)],
        out_shape=[_f32(N, HP), _f32(N, 8)],
    )(A1, xw1, hist1, b1p, p1rep)

    m1_2d, sel1_2d = pl.pallas_call(
        functools.partial(_select_body, K1),
        out_shape=[_f32(NROW2D, 128), _f32(NROW2D, 128)],
    )(_pad2d(sc1), ones2d)
    m1rep = _torep(m1_2d)
    sel1rep = _torep(sel1_2d)

    xw2, g1 = pl.pallas_call(
        _k3c_body,
        grid=(NB,),
        in_specs=[_rb(HP), _rb(8), _rb(8), _pb((HP, HP))],
        out_specs=[_rb(HP), _pb((2, HP))],
        out_shape=[_f32(N, HP), _f32(2, HP)],
    )(x1, sel1rep, m1rep, W2p)

    t2 = jnp.broadcast_to(m1_2d.reshape(NPAD)[:N, None], (N, WH))
    hist2 = _sc_gather_scatter_add(WH)(t2, srcp, dstp, zrows_h)

    xs2, dinv2rep = pl.pallas_call(
        _k4_body,
        grid=(NB,),
        in_specs=[_rb(HP), _hb(WH), _rb(8)],
        out_specs=[_rb(HP), _rb(8)],
        out_shape=[_f32(N, HP), _f32(N, 8)],
    )(xw2, hist2, m1rep)

    A2 = _sc_gather_scatter_add(HP)(xs2, srcp, dstp, zrows_f)

    x2, sc2 = pl.pallas_call(
        _k5a_body,
        grid=(NB,),
        in_specs=[_hb(HP), _rb(HP), _rb(8), _pb((1, HP)), _pb((HP, 8))],
        out_specs=[_rb(HP), _rb(8)],
        out_shape=[_f32(N, HP), _f32(N, 8)],
    )(A2, xw2, dinv2rep, b2p, p2rep)

    m2_2d, sel2_2d = pl.pallas_call(
        functools.partial(_select_body, K2),
        out_shape=[_f32(NROW2D, 128), _f32(NROW2D, 128)],
    )(_pad2d(sc2), m1_2d)

    g2 = pl.pallas_call(
        _k5c_body,
        grid=(NB,),
        in_specs=[_rb(HP), _rb(8), _rb(8)],
        out_specs=_pb((2, HP)),
        out_shape=_f32(2, HP),
    )(x2, _torep(sel2_2d), _torep(m2_2d))

    out = pl.pallas_call(
        _k6_body,
        out_shape=_f32(8, 128),
    )(g1, g2, wfc_pad, bfc_pad)
    return out[0:1, 0:NCLS]


# speedup vs baseline: 1.0163x; 1.0163x over previous
import functools
import math

import jax
import jax.numpy as jnp
from jax import lax
from jax.experimental import pallas as pl
from jax.experimental.pallas import tpu as pltpu
from jax.experimental.pallas import tpu_sc as plsc

N = 10000
E = 320000
F_IN = 128
HID = 50
HP = 64
WH = 16
NCLS = 4
K1 = int(math.ceil(0.5 * N))
K2 = int(math.ceil(0.5 * K1))
NPAD = 10240
NROW2D = NPAD // 128

NC = 2
NS = 16
NW = NC * NS
CH = 128
NCHUNK = E // CH
RPT = 632
ACCN = RPT * NS
KSS = 8
NCHUNKP = 2560
EP = NCHUNKP * CH
SSW = NCHUNKP // NW
NSSW = SSW // KSS

GB = 1000
NB = N // GB

_SC_MESH = plsc.VectorSubcoreMesh(core_axis_name="c", subcore_axis_name="s")


def _sc_gather_scatter_add(width):
    @functools.partial(
        pl.kernel,
        out_type=jax.ShapeDtypeStruct((NC, ACCN, width), jnp.float32),
        mesh=_SC_MESH,
        compiler_params=pltpu.CompilerParams(use_tc_tiling_on_sc=False),
        scratch_types=(
            [pltpu.VMEM((CH,), jnp.int32) for _ in range(2 * KSS)]
            + [
                pltpu.VMEM((KSS, CH, width), jnp.float32),
                pltpu.VMEM_SHARED((ACCN, width), jnp.float32),
                pltpu.SemaphoreType.DMA,
                pltpu.SemaphoreType.DMA,
            ]
        ),
    )
    def k(table, srcv, dstv, zrows, out, *scr):
        sidx = scr[0:KSS]
        didx = scr[KSS:2 * KSS]
        rows, acc, semg, sems = scr[2 * KSS:]
        c = lax.axis_index("c")
        s = lax.axis_index("s")
        wid = s * NC + c
        pltpu.sync_copy(zrows, acc.at[pl.ds(s * RPT, RPT)])
        plsc.subcore_barrier()
        base = wid * SSW * CH

        def step(i, carry):
            e0 = pl.multiple_of(base + i * (KSS * CH), CH)
            ld = [pltpu.async_copy(srcv.at[pl.ds(e0 + j * CH, CH)], sidx[j],
                                   semg) for j in range(KSS)]
            ld += [pltpu.async_copy(dstv.at[pl.ds(e0 + j * CH, CH)], didx[j],
                                    semg) for j in range(KSS)]
            for d in ld:
                d.wait()
            gd = [pltpu.async_copy(table.at[sidx[j]], rows.at[j], semg)
                  for j in range(KSS)]
            for d in gd:
                d.wait()
            sd = [pltpu.async_copy(rows.at[j], acc.at[didx[j]], sems,
                                   add=True) for j in range(KSS)]
            for d in sd:
                d.wait()
            return carry

        lax.fori_loop(0, NSSW, step, 0)
        plsc.subcore_barrier()
        pltpu.sync_copy(acc.at[pl.ds(s * RPT, RPT)],
                        out.at[c, pl.ds(s * RPT, RPT)])

    return k


def _sc_scatter_ones():
    @functools.partial(
        pl.kernel,
        out_type=jax.ShapeDtypeStruct((NC, ACCN, WH), jnp.float32),
        mesh=_SC_MESH,
        compiler_params=pltpu.CompilerParams(use_tc_tiling_on_sc=False),
        scratch_types=(
            [pltpu.VMEM((CH,), jnp.int32) for _ in range(KSS)]
            + [
                pltpu.VMEM((CH, WH), jnp.float32),
                pltpu.VMEM_SHARED((ACCN, WH), jnp.float32),
                pltpu.SemaphoreType.DMA,
                pltpu.SemaphoreType.DMA,
            ]
        ),
    )
    def k(dstv, onesr, zrows, out, *scr):
        didx = scr[0:KSS]
        rows, acc, semg, sems = scr[KSS:]
        c = lax.axis_index("c")
        s = lax.axis_index("s")
        wid = s * NC + c
        pltpu.sync_copy(zrows, acc.at[pl.ds(s * RPT, RPT)])
        pltpu.sync_copy(onesr, rows)
        plsc.subcore_barrier()
        base = wid * SSW * CH

        def step(i, carry):
            e0 = pl.multiple_of(base + i * (KSS * CH), CH)
            ld = [pltpu.async_copy(dstv.at[pl.ds(e0 + j * CH, CH)], didx[j],
                                   semg) for j in range(KSS)]
            for d in ld:
                d.wait()
            sd = [pltpu.async_copy(rows, acc.at[didx[j]], sems, add=True)
                  for j in range(KSS)]
            for d in sd:
                d.wait()
            return carry

        lax.fori_loop(0, NSSW, step, 0)
        plsc.subcore_barrier()
        pltpu.sync_copy(acc.at[pl.ds(s * RPT, RPT)],
                        out.at[c, pl.ds(s * RPT, RPT)])

    return k


def _k12_body(x_ref, w_ref, hist_ref, xw_ref, xs_ref):
    xw = jnp.dot(x_ref[...], w_ref[...], preferred_element_type=jnp.float32)
    d = hist_ref[0] + hist_ref[1]
    dinv = lax.rsqrt(1.0 + d[:, 0:1])
    xw_ref[...] = xw
    xs_ref[...] = xw * dinv


def _k3a_body(a_ref, xw_ref, hist_ref, b_ref, p_ref, x1_ref, sc_ref):
    d = hist_ref[0] + hist_ref[1]
    dinv = lax.rsqrt(1.0 + d[:, 0:1])
    agg = a_ref[0] + a_ref[1]
    xw = xw_ref[...]
    x1 = dinv * agg + (dinv * dinv) * xw + b_ref[...]
    x1_ref[...] = x1
    pn = p_ref[...]
    pnorm2 = jnp.sum(pn[:, 0:1] * pn[:, 0:1])
    sc = jnp.dot(x1, pn, preferred_element_type=jnp.float32) * lax.rsqrt(pnorm2)
    sc_ref[...] = jnp.tanh(sc)


def _select_body(k_const, sc_ref, msk_ref, m_ref, v_ref):
    sc = sc_ref[...]
    ridx = lax.broadcasted_iota(jnp.int32, (NROW2D, 128), 0)
    cidx = lax.broadcasted_iota(jnp.int32, (NROW2D, 128), 1)
    idx = ridx * 128 + cidx
    valid = (idx < N) & (msk_ref[...] > 0)
    bits = lax.bitcast_convert_type(sc, jnp.uint32)
    key = jnp.where((bits >> 31) == 0, bits | jnp.uint32(0x80000000), ~bits)
    key = jnp.where(valid, key, jnp.uint32(0))

    def tstep(b, t):
        sh = lax.shift_left(jnp.uint32(1), jnp.uint32(31) - b.astype(jnp.uint32))
        t_try = t | sh
        c = jnp.sum((key >= t_try).astype(jnp.int32))
        return jnp.where(c >= k_const, t_try, t)

    tthr = lax.fori_loop(0, 32, tstep, jnp.uint32(0))
    gt = key > tthr
    c_gt = jnp.sum(gt.astype(jnp.int32))
    r = k_const - c_gt
    eq = key == tthr

    def jstep(b, j):
        j_try = j | lax.shift_left(jnp.int32(1), jnp.int32(14) - b)
        c = jnp.sum((eq & (idx < j_try)).astype(jnp.int32))
        return jnp.where(c <= r, j_try, j)

    jcut = lax.fori_loop(0, 15, jstep, jnp.int32(0))
    sel = gt | (eq & (idx < jcut))
    m_ref[...] = sel.astype(jnp.float32)
    v_ref[...] = jnp.where(sel, sc, 0.0)


def _k3c_body(x1_ref, sel_ref, m_ref, w2_ref, xw2_ref, g_ref):
    i = pl.program_id(0)
    y = x1_ref[...] * sel_ref[...][:, 0:1]
    xw2_ref[...] = jnp.dot(y, w2_ref[...], preferred_element_type=jnp.float32)
    mcol = m_ref[...][:, 0:1] > 0
    bmax = jnp.max(jnp.where(mcol, y, -jnp.inf), axis=0, keepdims=True)
    bsum = jnp.sum(jnp.where(mcol, y, 0.0), axis=0, keepdims=True)

    @pl.when(i == 0)
    def _():
        g_ref[...] = jnp.concatenate([bmax, bsum], axis=0)

    @pl.when(i > 0)
    def _():
        prev = g_ref[...]
        g_ref[...] = jnp.concatenate(
            [jnp.maximum(prev[0:1], bmax), prev[1:2] + bsum], axis=0)

    @pl.when(i == NB - 1)
    def _():
        g = g_ref[...]
        g_ref[...] = jnp.concatenate([g[0:1], g[1:2] * (1.0 / K1)], axis=0)


def _k4_body(xw2_ref, hist_ref, m_ref, xs2_ref, dinv_ref):
    d = hist_ref[0] + hist_ref[1]
    mcol = m_ref[...][:, 0:1] > 0
    dinv = jnp.where(mcol, lax.rsqrt(1.0 + d[:, 0:1]), 0.0)
    xs2_ref[...] = xw2_ref[...] * dinv
    dinv_ref[...] = jnp.broadcast_to(dinv, (GB, 8))


def _k5a_body(a_ref, xw2_ref, dinv_ref, b_ref, p_ref, x2_ref, sc_ref):
    dinv = dinv_ref[...][:, 0:1]
    agg = a_ref[0] + a_ref[1]
    xw2 = xw2_ref[...]
    x2 = dinv * agg + (dinv * dinv) * xw2 + b_ref[...]
    x2_ref[...] = x2
    pn = p_ref[...]
    pnorm2 = jnp.sum(pn[:, 0:1] * pn[:, 0:1])
    sc = jnp.dot(x2, pn, preferred_element_type=jnp.float32) * lax.rsqrt(pnorm2)
    sc_ref[...] = jnp.tanh(sc)


def _k5c_body(x2_ref, sel_ref, m_ref, g_ref):
    i = pl.program_id(0)
    y = x2_ref[...] * sel_ref[...][:, 0:1]
    mcol = m_ref[...][:, 0:1] > 0
    bmax = jnp.max(jnp.where(mcol, y, -jnp.inf), axis=0, keepdims=True)
    bsum = jnp.sum(jnp.where(mcol, y, 0.0), axis=0, keepdims=True)

    @pl.when(i == 0)
    def _():
        g_ref[...] = jnp.concatenate([bmax, bsum], axis=0)

    @pl.when(i > 0)
    def _():
        prev = g_ref[...]
        g_ref[...] = jnp.concatenate(
            [jnp.maximum(prev[0:1], bmax), prev[1:2] + bsum], axis=0)

    @pl.when(i == NB - 1)
    def _():
        g = g_ref[...]
        g_ref[...] = jnp.concatenate([g[0:1], g[1:2] * (1.0 / K2)], axis=0)


def _k6_body(g1_ref, g2_ref, wfc_ref, bfc_ref, out_ref):
    dot = functools.partial(jnp.dot, preferred_element_type=jnp.float32)
    logits = (dot(g1_ref[0:1], wfc_ref[0]) + dot(g1_ref[1:2], wfc_ref[1])
              + dot(g2_ref[0:1], wfc_ref[2]) + dot(g2_ref[1:2], wfc_ref[3]))
    logits = logits + bfc_ref[...]
    col = lax.broadcasted_iota(jnp.int32, (1, 128), 1)
    neg = jnp.where(col < NCLS, logits, -jnp.inf)
    m = jnp.max(neg)
    e = jnp.where(col < NCLS, jnp.exp(logits - m), 0.0)
    lse = jnp.log(jnp.sum(e)) + m
    out_ref[...] = jnp.broadcast_to(logits - lse, (8, 128))


def _rb(width):
    return pl.BlockSpec((GB, width), lambda i: (i, 0))


def _pb(shape):
    return pl.BlockSpec(shape, lambda i: tuple(0 for _ in shape))


def _hb(width):
    return pl.BlockSpec((NC, GB, width), lambda i: (0, i, 0))


def _f32(*shape):
    return jax.ShapeDtypeStruct(shape, jnp.float32)


def _pad2d(flat8):
    return jnp.pad(flat8[:, 0], (0, NPAD - N)).reshape(NROW2D, 128)


def _torep(arr2d):
    flat = arr2d.reshape(NPAD)[:N]
    return jnp.broadcast_to(flat[:, None], (N, 8))


def kernel(x, edge_index, batch, W1, b1, W2, b2, p1, p2, Wfc, bfc):
    f32 = jnp.float32
    src = edge_index[0].astype(jnp.int32)
    dst = edge_index[1].astype(jnp.int32)
    sink = 10016 + (jnp.arange(EP - E, dtype=jnp.int32) % 64)
    srcp = jnp.concatenate([src, jnp.zeros((EP - E,), jnp.int32)])
    dstp = jnp.concatenate([dst, sink])

    W1p = jnp.zeros((F_IN, HP), f32).at[:, :HID].set(W1)
    W2p = jnp.zeros((HP, HP), f32).at[:HID, :HID].set(W2)
    b1p = jnp.zeros((1, HP), f32).at[0, :HID].set(b1)
    b2p = jnp.zeros((1, HP), f32).at[0, :HID].set(b2)
    p1rep = jnp.broadcast_to(
        jnp.zeros((HP,), f32).at[:HID].set(p1)[:, None], (HP, 8))
    p2rep = jnp.broadcast_to(
        jnp.zeros((HP,), f32).at[:HID].set(p2)[:, None], (HP, 8))
    wfc_pad = jnp.zeros((4, HP, 128), f32)
    for blk in range(4):
        wfc_pad = wfc_pad.at[blk, :HID, :NCLS].set(Wfc[blk * HID:(blk + 1) * HID])
    bfc_pad = jnp.zeros((1, 128), f32).at[0, :NCLS].set(bfc)
    zrows_h = jnp.zeros((RPT, WH), f32)
    zrows_f = jnp.zeros((RPT, HP), f32)
    ones_r = jnp.ones((CH, WH), f32)
    ones2d = jnp.ones((NROW2D, 128), f32)

    hist1 = _sc_scatter_ones()(dstp, ones_r, zrows_h)

    xw1, xs1 = pl.pallas_call(
        _k12_body,
        grid=(NB,),
        in_specs=[_rb(F_IN), _pb((F_IN, HP)), _hb(WH)],
        out_specs=[_rb(HP), _rb(HP)],
        out_shape=[_f32(N, HP), _f32(N, HP)],
    )(x, W1p, hist1)

    A1 = _sc_gather_scatter_add(HP)(xs1, srcp, dstp, zrows_f)

    x1, sc1 = pl.pallas_call(
        _k3a_body,
        grid=(NB,),
        in_specs=[_hb(HP), _rb(HP), _hb(WH), _pb((1, HP)), _pb((HP, 8))],
        out_specs=[_rb(HP), _rb(8)],
        out_shape=[_f32(N, HP), _f32(N, 8)],
    )(A1, xw1, hist1, b1p, p1rep)

    m1_2d, sel1_2d = pl.pallas_call(
        functools.partial(_select_body, K1),
        out_shape=[_f32(NROW2D, 128), _f32(NROW2D, 128)],
    )(_pad2d(sc1), ones2d)
    m1rep = _torep(m1_2d)
    sel1rep = _torep(sel1_2d)

    xw2, g1 = pl.pallas_call(
        _k3c_body,
        grid=(NB,),
        in_specs=[_rb(HP), _rb(8), _rb(8), _pb((HP, HP))],
        out_specs=[_rb(HP), _pb((2, HP))],
        out_shape=[_f32(N, HP), _f32(2, HP)],
    )(x1, sel1rep, m1rep, W2p)

    t2 = jnp.broadcast_to(m1_2d.reshape(NPAD)[:N, None], (N, WH))
    hist2 = _sc_gather_scatter_add(WH)(t2, srcp, dstp, zrows_h)

    xs2, dinv2rep = pl.pallas_call(
        _k4_body,
        grid=(NB,),
        in_specs=[_rb(HP), _hb(WH), _rb(8)],
        out_specs=[_rb(HP), _rb(8)],
        out_shape=[_f32(N, HP), _f32(N, 8)],
    )(xw2, hist2, m1rep)

    A2 = _sc_gather_scatter_add(HP)(xs2, srcp, dstp, zrows_f)

    x2, sc2 = pl.pallas_call(
        _k5a_body,
        grid=(NB,),
        in_specs=[_hb(HP), _rb(HP), _rb(8), _pb((1, HP)), _pb((HP, 8))],
        out_specs=[_rb(HP), _rb(8)],
        out_shape=[_f32(N, HP), _f32(N, 8)],
    )(A2, xw2, dinv2rep, b2p, p2rep)

    m2_2d, sel2_2d = pl.pallas_call(
        functools.partial(_select_body, K2),
        out_shape=[_f32(NROW2D, 128), _f32(NROW2D, 128)],
    )(_pad2d(sc2), m1_2d)

    g2 = pl.pallas_call(
        _k5c_body,
        grid=(NB,),
        in_specs=[_rb(HP), _rb(8), _rb(8)],
        out_specs=_pb((2, HP)),
        out_shape=_f32(2, HP),
    )(x2, _torep(sel2_2d), _torep(m2_2d))

    out = pl.pallas_call(
        _k6_body,
        out_shape=_f32(8, 128),
    )(g1, g2, wfc_pad, bfc_pad)
    return out[0:1, 0:NCLS]


# speedup vs baseline: 1.0167x; 1.0003x over previous
import functools
import math

import jax
import jax.numpy as jnp
from jax import lax
from jax.experimental import pallas as pl
from jax.experimental.pallas import tpu as pltpu
from jax.experimental.pallas import tpu_sc as plsc

N = 10000
E = 320000
F_IN = 128
HID = 50
HP = 64
WH = 16
NCLS = 4
K1 = int(math.ceil(0.5 * N))
K2 = int(math.ceil(0.5 * K1))
NPAD = 10240
NROW2D = NPAD // 128

NC = 2
NS = 16
NW = NC * NS
CH = 256
NCHUNK = E // CH
RPT = 632
ACCN = RPT * NS
KSS = 4
NCHUNKP = 1280
EP = NCHUNKP * CH
SSW = NCHUNKP // NW
NSSW = SSW // KSS


GB = 1000
NB = N // GB

_SC_MESH = plsc.VectorSubcoreMesh(core_axis_name="c", subcore_axis_name="s")


def _sc_gather_scatter_add(width):
    @functools.partial(
        pl.kernel,
        out_type=jax.ShapeDtypeStruct((NC, ACCN, width), jnp.float32),
        mesh=_SC_MESH,
        compiler_params=pltpu.CompilerParams(use_tc_tiling_on_sc=False),
        scratch_types=(
            [pltpu.VMEM((CH,), jnp.int32) for _ in range(2 * KSS)]
            + [
                pltpu.VMEM((KSS, CH, width), jnp.float32),
                pltpu.VMEM_SHARED((ACCN, width), jnp.float32),
                pltpu.SemaphoreType.DMA,
                pltpu.SemaphoreType.DMA,
            ]
        ),
    )
    def k(table, srcv, dstv, zrows, out, *scr):
        sidx = scr[0:KSS]
        didx = scr[KSS:2 * KSS]
        rows, acc, semg, sems = scr[2 * KSS:]
        c = lax.axis_index("c")
        s = lax.axis_index("s")
        wid = s * NC + c
        pltpu.sync_copy(zrows, acc.at[pl.ds(s * RPT, RPT)])
        plsc.subcore_barrier()
        base = wid * SSW * CH

        def step(i, carry):
            e0 = pl.multiple_of(base + i * (KSS * CH), CH)
            ld = [pltpu.async_copy(srcv.at[pl.ds(e0 + j * CH, CH)], sidx[j],
                                   semg) for j in range(KSS)]
            ld += [pltpu.async_copy(dstv.at[pl.ds(e0 + j * CH, CH)], didx[j],
                                    semg) for j in range(KSS)]
            for d in ld:
                d.wait()
            gd = [pltpu.async_copy(table.at[sidx[j]], rows.at[j], semg)
                  for j in range(KSS)]
            for d in gd:
                d.wait()
            sd = [pltpu.async_copy(rows.at[j], acc.at[didx[j]], sems,
                                   add=True) for j in range(KSS)]
            for d in sd:
                d.wait()
            return carry

        lax.fori_loop(0, NSSW, step, 0)
        plsc.subcore_barrier()
        pltpu.sync_copy(acc.at[pl.ds(s * RPT, RPT)],
                        out.at[c, pl.ds(s * RPT, RPT)])

    return k


def _sc_scatter_ones():
    @functools.partial(
        pl.kernel,
        out_type=jax.ShapeDtypeStruct((NC, ACCN, WH), jnp.float32),
        mesh=_SC_MESH,
        compiler_params=pltpu.CompilerParams(use_tc_tiling_on_sc=False),
        scratch_types=(
            [pltpu.VMEM((CH,), jnp.int32) for _ in range(KSS)]
            + [
                pltpu.VMEM((CH, WH), jnp.float32),
                pltpu.VMEM_SHARED((ACCN, WH), jnp.float32),
                pltpu.SemaphoreType.DMA,
                pltpu.SemaphoreType.DMA,
            ]
        ),
    )
    def k(dstv, onesr, zrows, out, *scr):
        didx = scr[0:KSS]
        rows, acc, semg, sems = scr[KSS:]
        c = lax.axis_index("c")
        s = lax.axis_index("s")
        wid = s * NC + c
        pltpu.sync_copy(zrows, acc.at[pl.ds(s * RPT, RPT)])
        pltpu.sync_copy(onesr, rows)
        plsc.subcore_barrier()
        base = wid * SSW * CH

        def step(i, carry):
            e0 = pl.multiple_of(base + i * (KSS * CH), CH)
            ld = [pltpu.async_copy(dstv.at[pl.ds(e0 + j * CH, CH)], didx[j],
                                   semg) for j in range(KSS)]
            for d in ld:
                d.wait()
            sd = [pltpu.async_copy(rows, acc.at[didx[j]], sems, add=True)
                  for j in range(KSS)]
            for d in sd:
                d.wait()
            return carry

        lax.fori_loop(0, NSSW, step, 0)
        plsc.subcore_barrier()
        pltpu.sync_copy(acc.at[pl.ds(s * RPT, RPT)],
                        out.at[c, pl.ds(s * RPT, RPT)])

    return k


def _k12_body(x_ref, w_ref, hist_ref, xw_ref, xs_ref):
    xw = jnp.dot(x_ref[...], w_ref[...], preferred_element_type=jnp.float32)
    d = hist_ref[0] + hist_ref[1]
    dinv = lax.rsqrt(1.0 + d[:, 0:1])
    xw_ref[...] = xw
    xs_ref[...] = xw * dinv


def _k3a_body(a_ref, xw_ref, hist_ref, b_ref, p_ref, x1_ref, sc_ref):
    d = hist_ref[0] + hist_ref[1]
    dinv = lax.rsqrt(1.0 + d[:, 0:1])
    agg = a_ref[0] + a_ref[1]
    xw = xw_ref[...]
    x1 = dinv * agg + (dinv * dinv) * xw + b_ref[...]
    x1_ref[...] = x1
    pn = p_ref[...]
    pnorm2 = jnp.sum(pn[:, 0:1] * pn[:, 0:1])
    sc = jnp.dot(x1, pn, preferred_element_type=jnp.float32) * lax.rsqrt(pnorm2)
    sc_ref[...] = jnp.tanh(sc)


def _select_body(k_const, sc_ref, msk_ref, m_ref, v_ref):
    sc = sc_ref[...]
    ridx = lax.broadcasted_iota(jnp.int32, (NROW2D, 128), 0)
    cidx = lax.broadcasted_iota(jnp.int32, (NROW2D, 128), 1)
    idx = ridx * 128 + cidx
    valid = (idx < N) & (msk_ref[...] > 0)
    bits = lax.bitcast_convert_type(sc, jnp.uint32)
    key = jnp.where((bits >> 31) == 0, bits | jnp.uint32(0x80000000), ~bits)
    key = jnp.where(valid, key, jnp.uint32(0))

    def tstep(b, t):
        sh = lax.shift_left(jnp.uint32(1), jnp.uint32(31) - b.astype(jnp.uint32))
        t_try = t | sh
        c = jnp.sum((key >= t_try).astype(jnp.int32))
        return jnp.where(c >= k_const, t_try, t)

    tthr = lax.fori_loop(0, 32, tstep, jnp.uint32(0))
    gt = key > tthr
    c_gt = jnp.sum(gt.astype(jnp.int32))
    r = k_const - c_gt
    eq = key == tthr

    def jstep(b, j):
        j_try = j | lax.shift_left(jnp.int32(1), jnp.int32(14) - b)
        c = jnp.sum((eq & (idx < j_try)).astype(jnp.int32))
        return jnp.where(c <= r, j_try, j)

    jcut = lax.fori_loop(0, 15, jstep, jnp.int32(0))
    sel = gt | (eq & (idx < jcut))
    m_ref[...] = sel.astype(jnp.float32)
    v_ref[...] = jnp.where(sel, sc, 0.0)


def _k3c_body(x1_ref, sel_ref, m_ref, w2_ref, xw2_ref, g_ref):
    i = pl.program_id(0)
    y = x1_ref[...] * sel_ref[...][:, 0:1]
    xw2_ref[...] = jnp.dot(y, w2_ref[...], preferred_element_type=jnp.float32)
    mcol = m_ref[...][:, 0:1] > 0
    bmax = jnp.max(jnp.where(mcol, y, -jnp.inf), axis=0, keepdims=True)
    bsum = jnp.sum(jnp.where(mcol, y, 0.0), axis=0, keepdims=True)

    @pl.when(i == 0)
    def _():
        g_ref[...] = jnp.concatenate([bmax, bsum], axis=0)

    @pl.when(i > 0)
    def _():
        prev = g_ref[...]
        g_ref[...] = jnp.concatenate(
            [jnp.maximum(prev[0:1], bmax), prev[1:2] + bsum], axis=0)

    @pl.when(i == NB - 1)
    def _():
        g = g_ref[...]
        g_ref[...] = jnp.concatenate([g[0:1], g[1:2] * (1.0 / K1)], axis=0)


def _k4_body(xw2_ref, hist_ref, m_ref, xs2_ref, dinv_ref):
    d = hist_ref[0] + hist_ref[1]
    mcol = m_ref[...][:, 0:1] > 0
    dinv = jnp.where(mcol, lax.rsqrt(1.0 + d[:, 0:1]), 0.0)
    xs2_ref[...] = xw2_ref[...] * dinv
    dinv_ref[...] = jnp.broadcast_to(dinv, (GB, 8))


def _k5a_body(a_ref, xw2_ref, dinv_ref, b_ref, p_ref, x2_ref, sc_ref):
    dinv = dinv_ref[...][:, 0:1]
    agg = a_ref[0] + a_ref[1]
    xw2 = xw2_ref[...]
    x2 = dinv * agg + (dinv * dinv) * xw2 + b_ref[...]
    x2_ref[...] = x2
    pn = p_ref[...]
    pnorm2 = jnp.sum(pn[:, 0:1] * pn[:, 0:1])
    sc = jnp.dot(x2, pn, preferred_element_type=jnp.float32) * lax.rsqrt(pnorm2)
    sc_ref[...] = jnp.tanh(sc)


def _k5c_body(x2_ref, sel_ref, m_ref, g_ref):
    i = pl.program_id(0)
    y = x2_ref[...] * sel_ref[...][:, 0:1]
    mcol = m_ref[...][:, 0:1] > 0
    bmax = jnp.max(jnp.where(mcol, y, -jnp.inf), axis=0, keepdims=True)
    bsum = jnp.sum(jnp.where(mcol, y, 0.0), axis=0, keepdims=True)

    @pl.when(i == 0)
    def _():
        g_ref[...] = jnp.concatenate([bmax, bsum], axis=0)

    @pl.when(i > 0)
    def _():
        prev = g_ref[...]
        g_ref[...] = jnp.concatenate(
            [jnp.maximum(prev[0:1], bmax), prev[1:2] + bsum], axis=0)

    @pl.when(i == NB - 1)
    def _():
        g = g_ref[...]
        g_ref[...] = jnp.concatenate([g[0:1], g[1:2] * (1.0 / K2)], axis=0)


def _k6_body(g1_ref, g2_ref, wfc_ref, bfc_ref, out_ref):
    dot = functools.partial(jnp.dot, preferred_element_type=jnp.float32)
    logits = (dot(g1_ref[0:1], wfc_ref[0]) + dot(g1_ref[1:2], wfc_ref[1])
              + dot(g2_ref[0:1], wfc_ref[2]) + dot(g2_ref[1:2], wfc_ref[3]))
    logits = logits + bfc_ref[...]
    col = lax.broadcasted_iota(jnp.int32, (1, 128), 1)
    neg = jnp.where(col < NCLS, logits, -jnp.inf)
    m = jnp.max(neg)
    e = jnp.where(col < NCLS, jnp.exp(logits - m), 0.0)
    lse = jnp.log(jnp.sum(e)) + m
    out_ref[...] = jnp.broadcast_to(logits - lse, (8, 128))


def _rb(width):
    return pl.BlockSpec((GB, width), lambda i: (i, 0))


def _pb(shape):
    return pl.BlockSpec(shape, lambda i: tuple(0 for _ in shape))


def _hb(width):
    return pl.BlockSpec((NC, GB, width), lambda i: (0, i, 0))


def _f32(*shape):
    return jax.ShapeDtypeStruct(shape, jnp.float32)


def _pad2d(flat8):
    return jnp.pad(flat8[:, 0], (0, NPAD - N)).reshape(NROW2D, 128)


def _torep(arr2d):
    flat = arr2d.reshape(NPAD)[:N]
    return jnp.broadcast_to(flat[:, None], (N, 8))


def kernel(x, edge_index, batch, W1, b1, W2, b2, p1, p2, Wfc, bfc):
    f32 = jnp.float32
    src = edge_index[0].astype(jnp.int32)
    dst = edge_index[1].astype(jnp.int32)
    sink = 10016 + (jnp.arange(EP - E, dtype=jnp.int32) % 64)
    srcp = jnp.concatenate([src, jnp.zeros((EP - E,), jnp.int32)])
    dstp = jnp.concatenate([dst, sink])

    W1p = jnp.zeros((F_IN, HP), f32).at[:, :HID].set(W1)
    W2p = jnp.zeros((HP, HP), f32).at[:HID, :HID].set(W2)
    b1p = jnp.zeros((1, HP), f32).at[0, :HID].set(b1)
    b2p = jnp.zeros((1, HP), f32).at[0, :HID].set(b2)
    p1rep = jnp.broadcast_to(
        jnp.zeros((HP,), f32).at[:HID].set(p1)[:, None], (HP, 8))
    p2rep = jnp.broadcast_to(
        jnp.zeros((HP,), f32).at[:HID].set(p2)[:, None], (HP, 8))
    wfc_pad = jnp.zeros((4, HP, 128), f32)
    for blk in range(4):
        wfc_pad = wfc_pad.at[blk, :HID, :NCLS].set(Wfc[blk * HID:(blk + 1) * HID])
    bfc_pad = jnp.zeros((1, 128), f32).at[0, :NCLS].set(bfc)
    zrows_h = jnp.zeros((RPT, WH), f32)
    zrows_f = jnp.zeros((RPT, HP), f32)
    ones_r = jnp.ones((CH, WH), f32)
    ones2d = jnp.ones((NROW2D, 128), f32)

    hist1 = _sc_scatter_ones()(dstp, ones_r, zrows_h)

    xw1, xs1 = pl.pallas_call(
        _k12_body,
        grid=(NB,),
        in_specs=[_rb(F_IN), _pb((F_IN, HP)), _hb(WH)],
        out_specs=[_rb(HP), _rb(HP)],
        out_shape=[_f32(N, HP), _f32(N, HP)],
    )(x, W1p, hist1)

    A1 = _sc_gather_scatter_add(HP)(xs1, srcp, dstp, zrows_f)

    x1, sc1 = pl.pallas_call(
        _k3a_body,
        grid=(NB,),
        in_specs=[_hb(HP), _rb(HP), _hb(WH), _pb((1, HP)), _pb((HP, 8))],
        out_specs=[_rb(HP), _rb(8)],
        out_shape=[_f32(N, HP), _f32(N, 8)],
    )(A1, xw1, hist1, b1p, p1rep)

    m1_2d, sel1_2d = pl.pallas_call(
        functools.partial(_select_body, K1),
        out_shape=[_f32(NROW2D, 128), _f32(NROW2D, 128)],
    )(_pad2d(sc1), ones2d)
    m1rep = _torep(m1_2d)
    sel1rep = _torep(sel1_2d)

    xw2, g1 = pl.pallas_call(
        _k3c_body,
        grid=(NB,),
        in_specs=[_rb(HP), _rb(8), _rb(8), _pb((HP, HP))],
        out_specs=[_rb(HP), _pb((2, HP))],
        out_shape=[_f32(N, HP), _f32(2, HP)],
    )(x1, sel1rep, m1rep, W2p)

    t2 = jnp.broadcast_to(m1_2d.reshape(NPAD)[:N, None], (N, WH))
    hist2 = _sc_gather_scatter_add(WH)(t2, srcp, dstp, zrows_h)

    xs2, dinv2rep = pl.pallas_call(
        _k4_body,
        grid=(NB,),
        in_specs=[_rb(HP), _hb(WH), _rb(8)],
        out_specs=[_rb(HP), _rb(8)],
        out_shape=[_f32(N, HP), _f32(N, 8)],
    )(xw2, hist2, m1rep)

    A2 = _sc_gather_scatter_add(HP)(xs2, srcp, dstp, zrows_f)

    x2, sc2 = pl.pallas_call(
        _k5a_body,
        grid=(NB,),
        in_specs=[_hb(HP), _rb(HP), _rb(8), _pb((1, HP)), _pb((HP, 8))],
        out_specs=[_rb(HP), _rb(8)],
        out_shape=[_f32(N, HP), _f32(N, 8)],
    )(A2, xw2, dinv2rep, b2p, p2rep)

    m2_2d, sel2_2d = pl.pallas_call(
        functools.partial(_select_body, K2),
        out_shape=[_f32(NROW2D, 128), _f32(NROW2D, 128)],
    )(_pad2d(sc2), m1_2d)

    g2 = pl.pallas_call(
        _k5c_body,
        grid=(NB,),
        in_specs=[_rb(HP), _rb(8), _rb(8)],
        out_specs=_pb((2, HP)),
        out_shape=_f32(2, HP),
    )(x2, _torep(sel2_2d), _torep(m2_2d))

    out = pl.pallas_call(
        _k6_body,
        out_shape=_f32(8, 128),
    )(g1, g2, wfc_pad, bfc_pad)
    return out[0:1, 0:NCLS]


# speedup vs baseline: 1.1067x; 1.0885x over previous
import functools
import math

import jax
import jax.numpy as jnp
from jax import lax
from jax.experimental import pallas as pl
from jax.experimental.pallas import tpu as pltpu
from jax.experimental.pallas import tpu_sc as plsc

N = 10000
E = 320000
F_IN = 128
HID = 50
HP = 64
WH = 16
NCLS = 4
K1 = int(math.ceil(0.5 * N))
K2 = int(math.ceil(0.5 * K1))
NPAD = 10240
NROW2D = NPAD // 128

NC = 2
NS = 16
NW = NC * NS
CH = 256
NCHUNK = E // CH
RPT = 632
ACCN = RPT * NS
KSS = 4
NCHUNKP = 1280
EP = NCHUNKP * CH
SSW = NCHUNKP // NW
NSSW = SSW // KSS


GB = 1000
NB = N // GB

_SC_MESH = plsc.VectorSubcoreMesh(core_axis_name="c", subcore_axis_name="s")


def _sc_gather_scatter_add(width):
    @functools.partial(
        pl.kernel,
        out_type=jax.ShapeDtypeStruct((NC, ACCN, width), jnp.float32),
        mesh=_SC_MESH,
        compiler_params=pltpu.CompilerParams(use_tc_tiling_on_sc=False),
        scratch_types=(
            [pltpu.VMEM((CH,), jnp.int32) for _ in range(2 * KSS)]
            + [
                pltpu.VMEM((KSS, CH, width), jnp.float32),
                pltpu.VMEM_SHARED((ACCN, width), jnp.float32),
                pltpu.SemaphoreType.DMA,
                pltpu.SemaphoreType.DMA,
            ]
        ),
    )
    def k(table, srcv, dstv, zrows, out, *scr):
        sidx = scr[0:KSS]
        didx = scr[KSS:2 * KSS]
        rows, acc, semg, sems = scr[2 * KSS:]
        c = lax.axis_index("c")
        s = lax.axis_index("s")
        wid = s * NC + c
        pltpu.sync_copy(zrows, acc.at[pl.ds(s * RPT, RPT)])
        plsc.subcore_barrier()
        base = wid * CH

        def step(i, carry):
            e0 = pl.multiple_of(base + i * (KSS * NW * CH), CH)
            ld = [pltpu.async_copy(srcv.at[pl.ds(e0 + j * (NW * CH), CH)],
                                   sidx[j], semg) for j in range(KSS)]
            ld += [pltpu.async_copy(dstv.at[pl.ds(e0 + j * (NW * CH), CH)],
                                    didx[j], semg) for j in range(KSS)]
            for d in ld:
                d.wait()
            gd = [pltpu.async_copy(table.at[sidx[j]], rows.at[j], semg)
                  for j in range(KSS)]
            for d in gd:
                d.wait()
            sd = [pltpu.async_copy(rows.at[j], acc.at[didx[j]], sems,
                                   add=True) for j in range(KSS)]
            for d in sd:
                d.wait()
            return carry

        lax.fori_loop(0, NSSW, step, 0)
        plsc.subcore_barrier()
        pltpu.sync_copy(acc.at[pl.ds(s * RPT, RPT)],
                        out.at[c, pl.ds(s * RPT, RPT)])

    return k


def _sc_scatter_ones():
    @functools.partial(
        pl.kernel,
        out_type=jax.ShapeDtypeStruct((NC, ACCN, WH), jnp.float32),
        mesh=_SC_MESH,
        compiler_params=pltpu.CompilerParams(use_tc_tiling_on_sc=False),
        scratch_types=(
            [pltpu.VMEM((CH,), jnp.int32) for _ in range(KSS)]
            + [
                pltpu.VMEM((CH, WH), jnp.float32),
                pltpu.VMEM_SHARED((ACCN, WH), jnp.float32),
                pltpu.SemaphoreType.DMA,
                pltpu.SemaphoreType.DMA,
            ]
        ),
    )
    def k(dstv, onesr, zrows, out, *scr):
        didx = scr[0:KSS]
        rows, acc, semg, sems = scr[KSS:]
        c = lax.axis_index("c")
        s = lax.axis_index("s")
        wid = s * NC + c
        pltpu.sync_copy(zrows, acc.at[pl.ds(s * RPT, RPT)])
        pltpu.sync_copy(onesr, rows)
        plsc.subcore_barrier()
        base = wid * CH

        def step(i, carry):
            e0 = pl.multiple_of(base + i * (KSS * NW * CH), CH)
            ld = [pltpu.async_copy(dstv.at[pl.ds(e0 + j * (NW * CH), CH)],
                                   didx[j], semg) for j in range(KSS)]
            for d in ld:
                d.wait()
            sd = [pltpu.async_copy(rows, acc.at[didx[j]], sems, add=True)
                  for j in range(KSS)]
            for d in sd:
                d.wait()
            return carry

        lax.fori_loop(0, NSSW, step, 0)
        plsc.subcore_barrier()
        pltpu.sync_copy(acc.at[pl.ds(s * RPT, RPT)],
                        out.at[c, pl.ds(s * RPT, RPT)])

    return k


def _k12_body(x_ref, w_ref, hist_ref, xw_ref, xs_ref):
    xw = jnp.dot(x_ref[...], w_ref[...], preferred_element_type=jnp.float32)
    d = hist_ref[0] + hist_ref[1]
    dinv = lax.rsqrt(1.0 + d[:, 0:1])
    xw_ref[...] = xw
    xs_ref[...] = xw * dinv


def _k3a_body(a_ref, xw_ref, hist_ref, b_ref, p_ref, x1_ref, sc_ref):
    d = hist_ref[0] + hist_ref[1]
    dinv = lax.rsqrt(1.0 + d[:, 0:1])
    agg = a_ref[0] + a_ref[1]
    xw = xw_ref[...]
    x1 = dinv * agg + (dinv * dinv) * xw + b_ref[...]
    x1_ref[...] = x1
    pn = p_ref[...]
    pnorm2 = jnp.sum(pn[:, 0:1] * pn[:, 0:1])
    sc = jnp.dot(x1, pn, preferred_element_type=jnp.float32) * lax.rsqrt(pnorm2)
    sc_ref[...] = jnp.tanh(sc)


def _select_body(k_const, sc_ref, msk_ref, m_ref, v_ref):
    sc = sc_ref[...]
    ridx = lax.broadcasted_iota(jnp.int32, (NROW2D, 128), 0)
    cidx = lax.broadcasted_iota(jnp.int32, (NROW2D, 128), 1)
    idx = ridx * 128 + cidx
    valid = (idx < N) & (msk_ref[...] > 0)
    bits = lax.bitcast_convert_type(sc, jnp.uint32)
    key = jnp.where((bits >> 31) == 0, bits | jnp.uint32(0x80000000), ~bits)
    key = jnp.where(valid, key, jnp.uint32(0))

    def tstep(b, t):
        sh = lax.shift_left(jnp.uint32(1), jnp.uint32(31) - b.astype(jnp.uint32))
        t_try = t | sh
        c = jnp.sum((key >= t_try).astype(jnp.int32))
        return jnp.where(c >= k_const, t_try, t)

    tthr = lax.fori_loop(0, 32, tstep, jnp.uint32(0))
    gt = key > tthr
    c_gt = jnp.sum(gt.astype(jnp.int32))
    r = k_const - c_gt
    eq = key == tthr

    def jstep(b, j):
        j_try = j | lax.shift_left(jnp.int32(1), jnp.int32(14) - b)
        c = jnp.sum((eq & (idx < j_try)).astype(jnp.int32))
        return jnp.where(c <= r, j_try, j)

    jcut = lax.fori_loop(0, 15, jstep, jnp.int32(0))
    sel = gt | (eq & (idx < jcut))
    m_ref[...] = sel.astype(jnp.float32)
    v_ref[...] = jnp.where(sel, sc, 0.0)


def _k3c_body(x1_ref, sel_ref, m_ref, w2_ref, xw2_ref, g_ref):
    i = pl.program_id(0)
    y = x1_ref[...] * sel_ref[...][:, 0:1]
    xw2_ref[...] = jnp.dot(y, w2_ref[...], preferred_element_type=jnp.float32)
    mcol = m_ref[...][:, 0:1] > 0
    bmax = jnp.max(jnp.where(mcol, y, -jnp.inf), axis=0, keepdims=True)
    bsum = jnp.sum(jnp.where(mcol, y, 0.0), axis=0, keepdims=True)

    @pl.when(i == 0)
    def _():
        g_ref[...] = jnp.concatenate([bmax, bsum], axis=0)

    @pl.when(i > 0)
    def _():
        prev = g_ref[...]
        g_ref[...] = jnp.concatenate(
            [jnp.maximum(prev[0:1], bmax), prev[1:2] + bsum], axis=0)

    @pl.when(i == NB - 1)
    def _():
        g = g_ref[...]
        g_ref[...] = jnp.concatenate([g[0:1], g[1:2] * (1.0 / K1)], axis=0)


def _k4_body(xw2_ref, hist_ref, m_ref, xs2_ref, dinv_ref):
    d = hist_ref[0] + hist_ref[1]
    mcol = m_ref[...][:, 0:1] > 0
    dinv = jnp.where(mcol, lax.rsqrt(1.0 + d[:, 0:1]), 0.0)
    xs2_ref[...] = xw2_ref[...] * dinv
    dinv_ref[...] = jnp.broadcast_to(dinv, (GB, 8))


def _k5a_body(a_ref, xw2_ref, dinv_ref, b_ref, p_ref, x2_ref, sc_ref):
    dinv = dinv_ref[...][:, 0:1]
    agg = a_ref[0] + a_ref[1]
    xw2 = xw2_ref[...]
    x2 = dinv * agg + (dinv * dinv) * xw2 + b_ref[...]
    x2_ref[...] = x2
    pn = p_ref[...]
    pnorm2 = jnp.sum(pn[:, 0:1] * pn[:, 0:1])
    sc = jnp.dot(x2, pn, preferred_element_type=jnp.float32) * lax.rsqrt(pnorm2)
    sc_ref[...] = jnp.tanh(sc)


def _k5c_body(x2_ref, sel_ref, m_ref, g_ref):
    i = pl.program_id(0)
    y = x2_ref[...] * sel_ref[...][:, 0:1]
    mcol = m_ref[...][:, 0:1] > 0
    bmax = jnp.max(jnp.where(mcol, y, -jnp.inf), axis=0, keepdims=True)
    bsum = jnp.sum(jnp.where(mcol, y, 0.0), axis=0, keepdims=True)

    @pl.when(i == 0)
    def _():
        g_ref[...] = jnp.concatenate([bmax, bsum], axis=0)

    @pl.when(i > 0)
    def _():
        prev = g_ref[...]
        g_ref[...] = jnp.concatenate(
            [jnp.maximum(prev[0:1], bmax), prev[1:2] + bsum], axis=0)

    @pl.when(i == NB - 1)
    def _():
        g = g_ref[...]
        g_ref[...] = jnp.concatenate([g[0:1], g[1:2] * (1.0 / K2)], axis=0)


def _k6_body(g1_ref, g2_ref, wfc_ref, bfc_ref, out_ref):
    dot = functools.partial(jnp.dot, preferred_element_type=jnp.float32)
    logits = (dot(g1_ref[0:1], wfc_ref[0]) + dot(g1_ref[1:2], wfc_ref[1])
              + dot(g2_ref[0:1], wfc_ref[2]) + dot(g2_ref[1:2], wfc_ref[3]))
    logits = logits + bfc_ref[...]
    col = lax.broadcasted_iota(jnp.int32, (1, 128), 1)
    neg = jnp.where(col < NCLS, logits, -jnp.inf)
    m = jnp.max(neg)
    e = jnp.where(col < NCLS, jnp.exp(logits - m), 0.0)
    lse = jnp.log(jnp.sum(e)) + m
    out_ref[...] = jnp.broadcast_to(logits - lse, (8, 128))


def _rb(width):
    return pl.BlockSpec((GB, width), lambda i: (i, 0))


def _pb(shape):
    return pl.BlockSpec(shape, lambda i: tuple(0 for _ in shape))


def _hb(width):
    return pl.BlockSpec((NC, GB, width), lambda i: (0, i, 0))


def _f32(*shape):
    return jax.ShapeDtypeStruct(shape, jnp.float32)


def _pad2d(flat8):
    return jnp.pad(flat8[:, 0], (0, NPAD - N)).reshape(NROW2D, 128)


def _torep(arr2d):
    flat = arr2d.reshape(NPAD)[:N]
    return jnp.broadcast_to(flat[:, None], (N, 8))


def kernel(x, edge_index, batch, W1, b1, W2, b2, p1, p2, Wfc, bfc):
    f32 = jnp.float32
    src = edge_index[0].astype(jnp.int32)
    dst = edge_index[1].astype(jnp.int32)
    sink = 10016 + (jnp.arange(EP - E, dtype=jnp.int32) % 64)
    srcp = jnp.concatenate([src, jnp.zeros((EP - E,), jnp.int32)])
    dstp = jnp.concatenate([dst, sink])

    W1p = jnp.zeros((F_IN, HP), f32).at[:, :HID].set(W1)
    W2p = jnp.zeros((HP, HP), f32).at[:HID, :HID].set(W2)
    b1p = jnp.zeros((1, HP), f32).at[0, :HID].set(b1)
    b2p = jnp.zeros((1, HP), f32).at[0, :HID].set(b2)
    p1rep = jnp.broadcast_to(
        jnp.zeros((HP,), f32).at[:HID].set(p1)[:, None], (HP, 8))
    p2rep = jnp.broadcast_to(
        jnp.zeros((HP,), f32).at[:HID].set(p2)[:, None], (HP, 8))
    wfc_pad = jnp.zeros((4, HP, 128), f32)
    for blk in range(4):
        wfc_pad = wfc_pad.at[blk, :HID, :NCLS].set(Wfc[blk * HID:(blk + 1) * HID])
    bfc_pad = jnp.zeros((1, 128), f32).at[0, :NCLS].set(bfc)
    zrows_h = jnp.zeros((RPT, WH), f32)
    zrows_f = jnp.zeros((RPT, HP), f32)
    ones_r = jnp.ones((CH, WH), f32)
    ones2d = jnp.ones((NROW2D, 128), f32)

    hist1 = _sc_scatter_ones()(dstp, ones_r, zrows_h)

    xw1, xs1 = pl.pallas_call(
        _k12_body,
        grid=(NB,),
        in_specs=[_rb(F_IN), _pb((F_IN, HP)), _hb(WH)],
        out_specs=[_rb(HP), _rb(HP)],
        out_shape=[_f32(N, HP), _f32(N, HP)],
    )(x, W1p, hist1)

    A1 = _sc_gather_scatter_add(HP)(xs1, srcp, dstp, zrows_f)

    x1, sc1 = pl.pallas_call(
        _k3a_body,
        grid=(NB,),
        in_specs=[_hb(HP), _rb(HP), _hb(WH), _pb((1, HP)), _pb((HP, 8))],
        out_specs=[_rb(HP), _rb(8)],
        out_shape=[_f32(N, HP), _f32(N, 8)],
    )(A1, xw1, hist1, b1p, p1rep)

    m1_2d, sel1_2d = pl.pallas_call(
        functools.partial(_select_body, K1),
        out_shape=[_f32(NROW2D, 128), _f32(NROW2D, 128)],
    )(_pad2d(sc1), ones2d)
    m1rep = _torep(m1_2d)
    sel1rep = _torep(sel1_2d)

    xw2, g1 = pl.pallas_call(
        _k3c_body,
        grid=(NB,),
        in_specs=[_rb(HP), _rb(8), _rb(8), _pb((HP, HP))],
        out_specs=[_rb(HP), _pb((2, HP))],
        out_shape=[_f32(N, HP), _f32(2, HP)],
    )(x1, sel1rep, m1rep, W2p)

    t2 = jnp.broadcast_to(m1_2d.reshape(NPAD)[:N, None], (N, WH))
    hist2 = _sc_gather_scatter_add(WH)(t2, srcp, dstp, zrows_h)

    xs2, dinv2rep = pl.pallas_call(
        _k4_body,
        grid=(NB,),
        in_specs=[_rb(HP), _hb(WH), _rb(8)],
        out_specs=[_rb(HP), _rb(8)],
        out_shape=[_f32(N, HP), _f32(N, 8)],
    )(xw2, hist2, m1rep)

    A2 = _sc_gather_scatter_add(HP)(xs2, srcp, dstp, zrows_f)

    x2, sc2 = pl.pallas_call(
        _k5a_body,
        grid=(NB,),
        in_specs=[_hb(HP), _rb(HP), _rb(8), _pb((1, HP)), _pb((HP, 8))],
        out_specs=[_rb(HP), _rb(8)],
        out_shape=[_f32(N, HP), _f32(N, 8)],
    )(A2, xw2, dinv2rep, b2p, p2rep)

    m2_2d, sel2_2d = pl.pallas_call(
        functools.partial(_select_body, K2),
        out_shape=[_f32(NROW2D, 128), _f32(NROW2D, 128)],
    )(_pad2d(sc2), m1_2d)

    g2 = pl.pallas_call(
        _k5c_body,
        grid=(NB,),
        in_specs=[_rb(HP), _rb(8), _rb(8)],
        out_specs=_pb((2, HP)),
        out_shape=_f32(2, HP),
    )(x2, _torep(sel2_2d), _torep(m2_2d))

    out = pl.pallas_call(
        _k6_body,
        out_shape=_f32(8, 128),
    )(g1, g2, wfc_pad, bfc_pad)
    return out[0:1, 0:NCLS]


# speedup vs baseline: 1.1118x; 1.0046x over previous
import functools
import math

import jax
import jax.numpy as jnp
from jax import lax
from jax.experimental import pallas as pl
from jax.experimental.pallas import tpu as pltpu
from jax.experimental.pallas import tpu_sc as plsc

N = 10000
E = 320000
F_IN = 128
HID = 50
HP = 64
WH = 16
NCLS = 4
K1 = int(math.ceil(0.5 * N))
K2 = int(math.ceil(0.5 * K1))
NPAD = 10240
NROW2D = NPAD // 128

NC = 2
NS = 16
NW = NC * NS
CH = 512
NCHUNK = E // CH
RPT = 632
ACCN = RPT * NS
KSS = 2
NCHUNKP = 640
EP = NCHUNKP * CH
SSW = NCHUNKP // NW
NSSW = SSW // KSS


GB = 1000
NB = N // GB

_SC_MESH = plsc.VectorSubcoreMesh(core_axis_name="c", subcore_axis_name="s")


def _sc_gather_scatter_add(width):
    @functools.partial(
        pl.kernel,
        out_type=jax.ShapeDtypeStruct((NC, ACCN, width), jnp.float32),
        mesh=_SC_MESH,
        compiler_params=pltpu.CompilerParams(use_tc_tiling_on_sc=False),
        scratch_types=(
            [pltpu.VMEM((CH,), jnp.int32) for _ in range(2 * KSS)]
            + [
                pltpu.VMEM((KSS, CH, width), jnp.float32),
                pltpu.VMEM_SHARED((ACCN, width), jnp.float32),
                pltpu.SemaphoreType.DMA,
                pltpu.SemaphoreType.DMA,
            ]
        ),
    )
    def k(table, srcv, dstv, zrows, out, *scr):
        sidx = scr[0:KSS]
        didx = scr[KSS:2 * KSS]
        rows, acc, semg, sems = scr[2 * KSS:]
        c = lax.axis_index("c")
        s = lax.axis_index("s")
        wid = s * NC + c
        pltpu.sync_copy(zrows, acc.at[pl.ds(s * RPT, RPT)])
        plsc.subcore_barrier()
        base = wid * CH

        def step(i, carry):
            e0 = pl.multiple_of(base + i * (KSS * NW * CH), CH)
            ld = [pltpu.async_copy(srcv.at[pl.ds(e0 + j * (NW * CH), CH)],
                                   sidx[j], semg) for j in range(KSS)]
            ld += [pltpu.async_copy(dstv.at[pl.ds(e0 + j * (NW * CH), CH)],
                                    didx[j], semg) for j in range(KSS)]
            for d in ld:
                d.wait()
            gd = [pltpu.async_copy(table.at[sidx[j]], rows.at[j], semg)
                  for j in range(KSS)]
            for d in gd:
                d.wait()
            sd = [pltpu.async_copy(rows.at[j], acc.at[didx[j]], sems,
                                   add=True) for j in range(KSS)]
            for d in sd:
                d.wait()
            return carry

        lax.fori_loop(0, NSSW, step, 0)
        plsc.subcore_barrier()
        pltpu.sync_copy(acc.at[pl.ds(s * RPT, RPT)],
                        out.at[c, pl.ds(s * RPT, RPT)])

    return k


def _sc_scatter_ones():
    @functools.partial(
        pl.kernel,
        out_type=jax.ShapeDtypeStruct((NC, ACCN, WH), jnp.float32),
        mesh=_SC_MESH,
        compiler_params=pltpu.CompilerParams(use_tc_tiling_on_sc=False),
        scratch_types=(
            [pltpu.VMEM((CH,), jnp.int32) for _ in range(KSS)]
            + [
                pltpu.VMEM((CH, WH), jnp.float32),
                pltpu.VMEM_SHARED((ACCN, WH), jnp.float32),
                pltpu.SemaphoreType.DMA,
                pltpu.SemaphoreType.DMA,
            ]
        ),
    )
    def k(dstv, onesr, zrows, out, *scr):
        didx = scr[0:KSS]
        rows, acc, semg, sems = scr[KSS:]
        c = lax.axis_index("c")
        s = lax.axis_index("s")
        wid = s * NC + c
        pltpu.sync_copy(zrows, acc.at[pl.ds(s * RPT, RPT)])
        pltpu.sync_copy(onesr, rows)
        plsc.subcore_barrier()
        base = wid * CH

        def step(i, carry):
            e0 = pl.multiple_of(base + i * (KSS * NW * CH), CH)
            ld = [pltpu.async_copy(dstv.at[pl.ds(e0 + j * (NW * CH), CH)],
                                   didx[j], semg) for j in range(KSS)]
            for d in ld:
                d.wait()
            sd = [pltpu.async_copy(rows, acc.at[didx[j]], sems, add=True)
                  for j in range(KSS)]
            for d in sd:
                d.wait()
            return carry

        lax.fori_loop(0, NSSW, step, 0)
        plsc.subcore_barrier()
        pltpu.sync_copy(acc.at[pl.ds(s * RPT, RPT)],
                        out.at[c, pl.ds(s * RPT, RPT)])

    return k


def _k12_body(x_ref, w_ref, hist_ref, xw_ref, xs_ref):
    xw = jnp.dot(x_ref[...], w_ref[...], preferred_element_type=jnp.float32)
    d = hist_ref[0] + hist_ref[1]
    dinv = lax.rsqrt(1.0 + d[:, 0:1])
    xw_ref[...] = xw
    xs_ref[...] = xw * dinv


def _k3a_body(a_ref, xw_ref, hist_ref, b_ref, p_ref, x1_ref, sc_ref):
    d = hist_ref[0] + hist_ref[1]
    dinv = lax.rsqrt(1.0 + d[:, 0:1])
    agg = a_ref[0] + a_ref[1]
    xw = xw_ref[...]
    x1 = dinv * agg + (dinv * dinv) * xw + b_ref[...]
    x1_ref[...] = x1
    pn = p_ref[...]
    pnorm2 = jnp.sum(pn[:, 0:1] * pn[:, 0:1])
    sc = jnp.dot(x1, pn, preferred_element_type=jnp.float32) * lax.rsqrt(pnorm2)
    sc_ref[...] = jnp.tanh(sc)


def _select_body(k_const, sc_ref, msk_ref, m_ref, v_ref):
    sc = sc_ref[...]
    ridx = lax.broadcasted_iota(jnp.int32, (NROW2D, 128), 0)
    cidx = lax.broadcasted_iota(jnp.int32, (NROW2D, 128), 1)
    idx = ridx * 128 + cidx
    valid = (idx < N) & (msk_ref[...] > 0)
    bits = lax.bitcast_convert_type(sc, jnp.uint32)
    key = jnp.where((bits >> 31) == 0, bits | jnp.uint32(0x80000000), ~bits)
    key = jnp.where(valid, key, jnp.uint32(0))

    def tstep(b, t):
        sh = lax.shift_left(jnp.uint32(1), jnp.uint32(31) - b.astype(jnp.uint32))
        t_try = t | sh
        c = jnp.sum((key >= t_try).astype(jnp.int32))
        return jnp.where(c >= k_const, t_try, t)

    tthr = lax.fori_loop(0, 32, tstep, jnp.uint32(0))
    gt = key > tthr
    c_gt = jnp.sum(gt.astype(jnp.int32))
    r = k_const - c_gt
    eq = key == tthr

    def jstep(b, j):
        j_try = j | lax.shift_left(jnp.int32(1), jnp.int32(14) - b)
        c = jnp.sum((eq & (idx < j_try)).astype(jnp.int32))
        return jnp.where(c <= r, j_try, j)

    jcut = lax.fori_loop(0, 15, jstep, jnp.int32(0))
    sel = gt | (eq & (idx < jcut))
    m_ref[...] = sel.astype(jnp.float32)
    v_ref[...] = jnp.where(sel, sc, 0.0)


def _k3c_body(x1_ref, sel_ref, m_ref, w2_ref, xw2_ref, g_ref):
    i = pl.program_id(0)
    y = x1_ref[...] * sel_ref[...][:, 0:1]
    xw2_ref[...] = jnp.dot(y, w2_ref[...], preferred_element_type=jnp.float32)
    mcol = m_ref[...][:, 0:1] > 0
    bmax = jnp.max(jnp.where(mcol, y, -jnp.inf), axis=0, keepdims=True)
    bsum = jnp.sum(jnp.where(mcol, y, 0.0), axis=0, keepdims=True)

    @pl.when(i == 0)
    def _():
        g_ref[...] = jnp.concatenate([bmax, bsum], axis=0)

    @pl.when(i > 0)
    def _():
        prev = g_ref[...]
        g_ref[...] = jnp.concatenate(
            [jnp.maximum(prev[0:1], bmax), prev[1:2] + bsum], axis=0)

    @pl.when(i == NB - 1)
    def _():
        g = g_ref[...]
        g_ref[...] = jnp.concatenate([g[0:1], g[1:2] * (1.0 / K1)], axis=0)


def _k4_body(xw2_ref, hist_ref, m_ref, xs2_ref, dinv_ref):
    d = hist_ref[0] + hist_ref[1]
    mcol = m_ref[...][:, 0:1] > 0
    dinv = jnp.where(mcol, lax.rsqrt(1.0 + d[:, 0:1]), 0.0)
    xs2_ref[...] = xw2_ref[...] * dinv
    dinv_ref[...] = jnp.broadcast_to(dinv, (GB, 8))


def _k5a_body(a_ref, xw2_ref, dinv_ref, b_ref, p_ref, x2_ref, sc_ref):
    dinv = dinv_ref[...][:, 0:1]
    agg = a_ref[0] + a_ref[1]
    xw2 = xw2_ref[...]
    x2 = dinv * agg + (dinv * dinv) * xw2 + b_ref[...]
    x2_ref[...] = x2
    pn = p_ref[...]
    pnorm2 = jnp.sum(pn[:, 0:1] * pn[:, 0:1])
    sc = jnp.dot(x2, pn, preferred_element_type=jnp.float32) * lax.rsqrt(pnorm2)
    sc_ref[...] = jnp.tanh(sc)


def _k5c_body(x2_ref, sel_ref, m_ref, g_ref):
    i = pl.program_id(0)
    y = x2_ref[...] * sel_ref[...][:, 0:1]
    mcol = m_ref[...][:, 0:1] > 0
    bmax = jnp.max(jnp.where(mcol, y, -jnp.inf), axis=0, keepdims=True)
    bsum = jnp.sum(jnp.where(mcol, y, 0.0), axis=0, keepdims=True)

    @pl.when(i == 0)
    def _():
        g_ref[...] = jnp.concatenate([bmax, bsum], axis=0)

    @pl.when(i > 0)
    def _():
        prev = g_ref[...]
        g_ref[...] = jnp.concatenate(
            [jnp.maximum(prev[0:1], bmax), prev[1:2] + bsum], axis=0)

    @pl.when(i == NB - 1)
    def _():
        g = g_ref[...]
        g_ref[...] = jnp.concatenate([g[0:1], g[1:2] * (1.0 / K2)], axis=0)


def _k6_body(g1_ref, g2_ref, wfc_ref, bfc_ref, out_ref):
    dot = functools.partial(jnp.dot, preferred_element_type=jnp.float32)
    logits = (dot(g1_ref[0:1], wfc_ref[0]) + dot(g1_ref[1:2], wfc_ref[1])
              + dot(g2_ref[0:1], wfc_ref[2]) + dot(g2_ref[1:2], wfc_ref[3]))
    logits = logits + bfc_ref[...]
    col = lax.broadcasted_iota(jnp.int32, (1, 128), 1)
    neg = jnp.where(col < NCLS, logits, -jnp.inf)
    m = jnp.max(neg)
    e = jnp.where(col < NCLS, jnp.exp(logits - m), 0.0)
    lse = jnp.log(jnp.sum(e)) + m
    out_ref[...] = jnp.broadcast_to(logits - lse, (8, 128))


def _rb(width):
    return pl.BlockSpec((GB, width), lambda i: (i, 0))


def _pb(shape):
    return pl.BlockSpec(shape, lambda i: tuple(0 for _ in shape))


def _hb(width):
    return pl.BlockSpec((NC, GB, width), lambda i: (0, i, 0))


def _f32(*shape):
    return jax.ShapeDtypeStruct(shape, jnp.float32)


def _pad2d(flat8):
    return jnp.pad(flat8[:, 0], (0, NPAD - N)).reshape(NROW2D, 128)


def _torep(arr2d):
    flat = arr2d.reshape(NPAD)[:N]
    return jnp.broadcast_to(flat[:, None], (N, 8))


def kernel(x, edge_index, batch, W1, b1, W2, b2, p1, p2, Wfc, bfc):
    f32 = jnp.float32
    src = edge_index[0].astype(jnp.int32)
    dst = edge_index[1].astype(jnp.int32)
    sink = 10016 + (jnp.arange(EP - E, dtype=jnp.int32) % 64)
    srcp = jnp.concatenate([src, jnp.zeros((EP - E,), jnp.int32)])
    dstp = jnp.concatenate([dst, sink])

    W1p = jnp.zeros((F_IN, HP), f32).at[:, :HID].set(W1)
    W2p = jnp.zeros((HP, HP), f32).at[:HID, :HID].set(W2)
    b1p = jnp.zeros((1, HP), f32).at[0, :HID].set(b1)
    b2p = jnp.zeros((1, HP), f32).at[0, :HID].set(b2)
    p1rep = jnp.broadcast_to(
        jnp.zeros((HP,), f32).at[:HID].set(p1)[:, None], (HP, 8))
    p2rep = jnp.broadcast_to(
        jnp.zeros((HP,), f32).at[:HID].set(p2)[:, None], (HP, 8))
    wfc_pad = jnp.zeros((4, HP, 128), f32)
    for blk in range(4):
        wfc_pad = wfc_pad.at[blk, :HID, :NCLS].set(Wfc[blk * HID:(blk + 1) * HID])
    bfc_pad = jnp.zeros((1, 128), f32).at[0, :NCLS].set(bfc)
    zrows_h = jnp.zeros((RPT, WH), f32)
    zrows_f = jnp.zeros((RPT, HP), f32)
    ones_r = jnp.ones((CH, WH), f32)
    ones2d = jnp.ones((NROW2D, 128), f32)

    hist1 = _sc_scatter_ones()(dstp, ones_r, zrows_h)

    xw1, xs1 = pl.pallas_call(
        _k12_body,
        grid=(NB,),
        in_specs=[_rb(F_IN), _pb((F_IN, HP)), _hb(WH)],
        out_specs=[_rb(HP), _rb(HP)],
        out_shape=[_f32(N, HP), _f32(N, HP)],
    )(x, W1p, hist1)

    A1 = _sc_gather_scatter_add(HP)(xs1, srcp, dstp, zrows_f)

    x1, sc1 = pl.pallas_call(
        _k3a_body,
        grid=(NB,),
        in_specs=[_hb(HP), _rb(HP), _hb(WH), _pb((1, HP)), _pb((HP, 8))],
        out_specs=[_rb(HP), _rb(8)],
        out_shape=[_f32(N, HP), _f32(N, 8)],
    )(A1, xw1, hist1, b1p, p1rep)

    m1_2d, sel1_2d = pl.pallas_call(
        functools.partial(_select_body, K1),
        out_shape=[_f32(NROW2D, 128), _f32(NROW2D, 128)],
    )(_pad2d(sc1), ones2d)
    m1rep = _torep(m1_2d)
    sel1rep = _torep(sel1_2d)

    xw2, g1 = pl.pallas_call(
        _k3c_body,
        grid=(NB,),
        in_specs=[_rb(HP), _rb(8), _rb(8), _pb((HP, HP))],
        out_specs=[_rb(HP), _pb((2, HP))],
        out_shape=[_f32(N, HP), _f32(2, HP)],
    )(x1, sel1rep, m1rep, W2p)

    t2 = jnp.broadcast_to(m1_2d.reshape(NPAD)[:N, None], (N, WH))
    hist2 = _sc_gather_scatter_add(WH)(t2, srcp, dstp, zrows_h)

    xs2, dinv2rep = pl.pallas_call(
        _k4_body,
        grid=(NB,),
        in_specs=[_rb(HP), _hb(WH), _rb(8)],
        out_specs=[_rb(HP), _rb(8)],
        out_shape=[_f32(N, HP), _f32(N, 8)],
    )(xw2, hist2, m1rep)

    A2 = _sc_gather_scatter_add(HP)(xs2, srcp, dstp, zrows_f)

    x2, sc2 = pl.pallas_call(
        _k5a_body,
        grid=(NB,),
        in_specs=[_hb(HP), _rb(HP), _rb(8), _pb((1, HP)), _pb((HP, 8))],
        out_specs=[_rb(HP), _rb(8)],
        out_shape=[_f32(N, HP), _f32(N, 8)],
    )(A2, xw2, dinv2rep, b2p, p2rep)

    m2_2d, sel2_2d = pl.pallas_call(
        functools.partial(_select_body, K2),
        out_shape=[_f32(NROW2D, 128), _f32(NROW2D, 128)],
    )(_pad2d(sc2), m1_2d)

    g2 = pl.pallas_call(
        _k5c_body,
        grid=(NB,),
        in_specs=[_rb(HP), _rb(8), _rb(8)],
        out_specs=_pb((2, HP)),
        out_shape=_f32(2, HP),
    )(x2, _torep(sel2_2d), _torep(m2_2d))

    out = pl.pallas_call(
        _k6_body,
        out_shape=_f32(8, 128),
    )(g1, g2, wfc_pad, bfc_pad)
    return out[0:1, 0:NCLS]
